# baseline (speedup 1.0000x reference)
import functools

import jax
import jax.numpy as jnp
from jax import lax
from jax.experimental import pallas as pl
from jax.experimental.pallas import tpu as pltpu
from jax.experimental.pallas import tpu_sc as plsc

_VSZ = 1000001
_DSZ = 64
_B = 4096
_L = 200

_ROW_BLK = 4096
_NBLK = -(-_VSZ // _ROW_BLK)
_NW = 32
_BPW = _B // _NW
_TPW = _BPW * _L
_GCHUNK = 128
_NCH = _TPW // _GCHUNK
_FIRE = 8


def _proj_body(lut_ref, w_ref, b_ref, pd_ref):
    w = w_ref[...]
    wd = w[1:2, :] - w[0:1, :]
    bd = b_ref[1] - b_ref[0]
    x = lut_ref[...]
    pd = lax.dot_general(wd, x, (((1,), (1,)), ((), ())),
                         preferred_element_type=jnp.float32)
    pd_ref[...] = (pd + bd).reshape(1, 1, _ROW_BLK)


def _project_table(lut_weight, out_weight, out_bias):
    return pl.pallas_call(
        _proj_body,
        grid=(_NBLK,),
        in_specs=[
            pl.BlockSpec((_ROW_BLK, _DSZ), lambda i: (i, 0)),
            pl.BlockSpec((2, _DSZ), lambda i: (0, 0)),
            pl.BlockSpec(memory_space=pltpu.SMEM),
        ],
        out_specs=pl.BlockSpec((1, 1, _ROW_BLK), lambda i: (i, 0, 0)),
        out_shape=jax.ShapeDtypeStruct((_NBLK, 1, _ROW_BLK), jnp.float32),
    )(lut_weight, out_weight, out_bias)


def _sc_body(pd_hbm, ids_hbm, d_hbm, idx_v, vals_v, out_v, sem):
    c = lax.axis_index("c")
    s = lax.axis_index("s")
    wid = s * 2 + c

    pltpu.sync_copy(ids_hbm.at[wid], idx_v)

    nacc = _BPW // 16

    def gather_group(g, accs):
        cps = []
        for u in range(_FIRE):
            j = g * _FIRE + u
            cps.append(pltpu.async_copy(
                pd_hbm.at[idx_v.at[j]],
                vals_v.at[pl.ds(j * _GCHUNK, _GCHUNK)],
                sem))
        for cp in cps:
            cp.wait()
        accs = list(accs)
        for u in range(_FIRE):
            base = (g * _FIRE + u) * _GCHUNK
            for r in range(nacc):
                accs[r] = accs[r] + vals_v[pl.ds(base + r * 16, 16)]
        return tuple(accs)

    accs = lax.fori_loop(
        0, _NCH // _FIRE, gather_group,
        tuple(jnp.zeros((16,), jnp.float32) for _ in range(nacc)),
        unroll=False)

    for r in range(nacc):
        out_v[pl.ds(r * 16, 16)] = accs[r] * (1.0 / _L)

    pltpu.sync_copy(out_v, d_hbm.at[pl.ds(wid * _BPW, _BPW)])


def _sc_gather_mean(pd_flat, ids3):
    mesh = plsc.VectorSubcoreMesh(core_axis_name="c", subcore_axis_name="s")
    run = pl.kernel(
        _sc_body,
        out_type=jax.ShapeDtypeStruct((_B,), jnp.float32),
        mesh=mesh,
        scratch_types=[
            pltpu.VMEM((_NCH, _GCHUNK), jnp.int32),
            pltpu.VMEM((_TPW,), jnp.float32),
            pltpu.VMEM((_BPW,), jnp.float32),
            pltpu.SemaphoreType.DMA,
        ],
    )
    return run(pd_flat, ids3)


def _head_body(d_ref, o0_ref, o1_ref):
    delta = d_ref[...]
    sp = jnp.maximum(delta, 0.0) + jnp.log1p(jnp.exp(-jnp.abs(delta)))
    o0_ref[...] = -sp
    o1_ref[...] = delta - sp


def _head(d2):
    return pl.pallas_call(
        _head_body,
        in_specs=[pl.BlockSpec((_NW, _BPW), lambda: (0, 0))],
        out_specs=[pl.BlockSpec((_NW, _BPW), lambda: (0, 0))] * 2,
        out_shape=[jax.ShapeDtypeStruct((_NW, _BPW), jnp.float32)] * 2,
    )(d2)


def kernel(input, lut_weight, out_weight, out_bias):
    ids = input.astype(jnp.int32)
    pd = _project_table(lut_weight, out_weight, out_bias).reshape(-1)
    ids3 = ids.reshape(_NW, _BPW, _L).transpose(0, 2, 1)
    delta = _sc_gather_mean(pd, ids3)
    o0, o1 = _head(delta.reshape(_NW, _BPW))
    return jnp.stack([o0.reshape(_B), o1.reshape(_B)], axis=-1)


# speedup vs baseline: 1.2221x; 1.2221x over previous
import functools

import jax
import jax.numpy as jnp
from jax import lax
from jax.experimental import pallas as pl
from jax.experimental.pallas import tpu as pltpu
from jax.experimental.pallas import tpu_sc as plsc

_VSZ = 1000001
_DSZ = 64
_B = 4096
_L = 200

_ROW_BLK = 16384
_NBLK = -(-_VSZ // _ROW_BLK)
_NW = 32
_BPW = _B // _NW
_TPW = _BPW * _L
_GCHUNK = 128
_NCH = _TPW // _GCHUNK
_FIRE = 8


def _proj_body(lut_ref, w_ref, b_ref, pd_ref):
    w = w_ref[...]
    wd = w[1:2, :] - w[0:1, :]
    bd = b_ref[1] - b_ref[0]
    x = lut_ref[...]
    pd = lax.dot_general(wd, x, (((1,), (1,)), ((), ())),
                         preferred_element_type=jnp.float32)
    pd_ref[...] = (pd + bd).reshape(1, 1, _ROW_BLK)


def _project_table(lut_weight, out_weight, out_bias):
    return pl.pallas_call(
        _proj_body,
        grid=(_NBLK,),
        in_specs=[
            pl.BlockSpec((_ROW_BLK, _DSZ), lambda i: (i, 0)),
            pl.BlockSpec((2, _DSZ), lambda i: (0, 0)),
            pl.BlockSpec(memory_space=pltpu.SMEM),
        ],
        out_specs=pl.BlockSpec((1, 1, _ROW_BLK), lambda i: (i, 0, 0)),
        out_shape=jax.ShapeDtypeStruct((_NBLK, 1, _ROW_BLK), jnp.float32),
    )(lut_weight, out_weight, out_bias)


def _sc_body(pd_hbm, ids_hbm, d_hbm, idx_v, vals_v, out_v, sem):
    c = lax.axis_index("c")
    s = lax.axis_index("s")
    wid = s * 2 + c

    pltpu.sync_copy(ids_hbm.at[wid], idx_v)

    nacc = _BPW // 16

    def issue(j, carry):
        pltpu.async_copy(
            pd_hbm.at[idx_v.at[j]],
            vals_v.at[pl.ds(j * _GCHUNK, _GCHUNK)],
            sem)
        return carry

    lax.fori_loop(0, _NCH, issue, 0, unroll=False)

    pltpu.make_async_copy(pd_hbm.at[pl.ds(0, _TPW)], vals_v, sem).wait()

    def acc_chunk(j, accs):
        base = j * _GCHUNK
        return tuple(
            accs[r] + vals_v[pl.ds(base + r * 16, 16)]
            for r in range(nacc))

    accs = lax.fori_loop(
        0, _NCH, acc_chunk,
        tuple(jnp.zeros((16,), jnp.float32) for _ in range(nacc)),
        unroll=False)

    for r in range(nacc):
        out_v[pl.ds(r * 16, 16)] = accs[r] * (1.0 / _L)

    pltpu.sync_copy(out_v, d_hbm.at[pl.ds(wid * _BPW, _BPW)])


def _sc_gather_mean(pd_flat, ids3):
    mesh = plsc.VectorSubcoreMesh(core_axis_name="c", subcore_axis_name="s")
    run = pl.kernel(
        _sc_body,
        out_type=jax.ShapeDtypeStruct((_B,), jnp.float32),
        mesh=mesh,
        scratch_types=[
            pltpu.VMEM((_NCH, _GCHUNK), jnp.int32),
            pltpu.VMEM((_TPW,), jnp.float32),
            pltpu.VMEM((_BPW,), jnp.float32),
            pltpu.SemaphoreType.DMA,
        ],
    )
    return run(pd_flat, ids3)


def _head_body(d_ref, o0_ref, o1_ref):
    delta = d_ref[...]
    sp = jnp.maximum(delta, 0.0) + jnp.log1p(jnp.exp(-jnp.abs(delta)))
    o0_ref[...] = -sp
    o1_ref[...] = delta - sp


def _head(d2):
    return pl.pallas_call(
        _head_body,
        in_specs=[pl.BlockSpec((_NW, _BPW), lambda: (0, 0))],
        out_specs=[pl.BlockSpec((_NW, _BPW), lambda: (0, 0))] * 2,
        out_shape=[jax.ShapeDtypeStruct((_NW, _BPW), jnp.float32)] * 2,
    )(d2)


def kernel(input, lut_weight, out_weight, out_bias):
    ids = input.astype(jnp.int32)
    pd = _project_table(lut_weight, out_weight, out_bias).reshape(-1)
    ids3 = ids.reshape(_NW, _BPW, _L).transpose(0, 2, 1)
    delta = _sc_gather_mean(pd, ids3)
    o0, o1 = _head(delta.reshape(_NW, _BPW))
    return jnp.stack([o0.reshape(_B), o1.reshape(_B)], axis=-1)
